# baseline (speedup 1.0000x reference)
import functools

import jax
import jax.numpy as jnp
from jax.experimental import pallas as pl
from jax.experimental.pallas import tpu as pltpu

_BN_EPS = 1e-5
_VMEM_LIMIT = 48 * 1024 * 1024


def _fused_body(x_ref, w_ref, b_ref, o_ref, acc_ref, *, n_ci, c_blk):
    ci = pl.program_id(1)

    @pl.when(ci == 0)
    def _():
        acc_ref[...] = jnp.zeros_like(acc_ref)

    sums = jnp.sum(x_ref[...], axis=(-2, -1), dtype=jnp.float32)
    w_blk = w_ref[pl.ds(ci * c_blk, c_blk), :]
    acc_ref[...] += jnp.dot(sums, w_blk, preferred_element_type=jnp.float32)

    @pl.when(ci == n_ci - 1)
    def _():
        act = jnp.maximum(acc_ref[...] + b_ref[...], 0.0).astype(o_ref.dtype)
        o_ref[...] = jnp.broadcast_to(act[:, :, None, None], o_ref.shape)


def _largest_divisor(n, target):
    d = min(n, target)
    while n % d:
        d -= 1
    return d


def kernel(x, conv_w, gamma, beta, running_mean, running_var):
    N, Cin, H, W = x.shape
    Cout = conv_w.shape[0]
    HW = H * W

    scale = gamma.astype(jnp.float32) * jax.lax.rsqrt(
        running_var.astype(jnp.float32) + _BN_EPS)
    w2d = conv_w.reshape(Cout, Cin).astype(jnp.float32)
    w_folded = (w2d * scale[:, None] / HW).T
    bias = (beta.astype(jnp.float32)
            - running_mean.astype(jnp.float32) * scale).reshape(1, Cout)

    S, L = H, W
    x4 = x

    nb_sz = _largest_divisor(N, 2)
    c_blk = _largest_divisor(Cin, 256)
    n_nb = N // nb_sz
    n_ci = Cin // c_blk
    itemsize = jnp.dtype(x.dtype).itemsize

    out4 = pl.pallas_call(
        functools.partial(_fused_body, n_ci=n_ci, c_blk=c_blk),
        out_shape=jax.ShapeDtypeStruct((N, Cout, S, L), x.dtype),
        grid=(n_nb, n_ci),
        in_specs=[
            pl.BlockSpec((nb_sz, c_blk, S, L), lambda nb, ci: (nb, ci, 0, 0)),
            pl.BlockSpec((Cin, Cout), lambda nb, ci: (0, 0)),
            pl.BlockSpec((1, Cout), lambda nb, ci: (0, 0)),
        ],
        out_specs=pl.BlockSpec((nb_sz, Cout, S, L),
                               lambda nb, ci: (nb, 0, 0, 0)),
        scratch_shapes=[pltpu.VMEM((nb_sz, Cout), jnp.float32)],
        compiler_params=pltpu.CompilerParams(
            dimension_semantics=("parallel", "arbitrary"),
            vmem_limit_bytes=_VMEM_LIMIT),
        cost_estimate=pl.CostEstimate(
            flops=N * Cin * HW + 2 * N * Cin * Cout,
            transcendentals=0,
            bytes_accessed=N * Cin * HW * itemsize
                           + N * Cout * HW * itemsize + Cin * Cout * 4),
    )(x4, w_folded, bias)

    return out4.reshape(N, Cout, H, W)


# speedup vs baseline: 10.8457x; 10.8457x over previous
import functools

import jax
import jax.numpy as jnp
from jax.experimental import pallas as pl
from jax.experimental.pallas import tpu as pltpu

_BN_EPS = 1e-5
_VMEM_LIMIT = 48 * 1024 * 1024


def _fused_body(x_ref, w_ref, b_ref, o_ref):
    pooled = jnp.sum(x_ref[...], axis=(1, 2), dtype=jnp.float32)
    y = jnp.dot(pooled, w_ref[...], preferred_element_type=jnp.float32)
    act = jnp.maximum(y + b_ref[...], 0.0).astype(o_ref.dtype)
    o_ref[...] = jnp.broadcast_to(act[:, None, None, :], o_ref.shape)


def _largest_divisor(n, target):
    d = min(n, target)
    while n % d:
        d -= 1
    return d


def kernel(x, conv_w, gamma, beta, running_mean, running_var):
    N, Cin, H, W = x.shape
    Cout = conv_w.shape[0]
    HW = H * W

    scale = gamma.astype(jnp.float32) * jax.lax.rsqrt(
        running_var.astype(jnp.float32) + _BN_EPS)
    w2d = conv_w.reshape(Cout, Cin).astype(jnp.float32)
    w_folded = (w2d * scale[:, None] / HW).T
    bias = (beta.astype(jnp.float32)
            - running_mean.astype(jnp.float32) * scale).reshape(1, Cout)

    xt = jnp.transpose(x, (0, 2, 3, 1))

    nb_sz = _largest_divisor(N, 1)
    n_nb = N // nb_sz
    itemsize = jnp.dtype(x.dtype).itemsize

    out_t = pl.pallas_call(
        _fused_body,
        out_shape=jax.ShapeDtypeStruct((N, H, W, Cout), x.dtype),
        grid=(n_nb,),
        in_specs=[
            pl.BlockSpec((nb_sz, H, W, Cin), lambda nb: (nb, 0, 0, 0)),
            pl.BlockSpec((Cin, Cout), lambda nb: (0, 0)),
            pl.BlockSpec((1, Cout), lambda nb: (0, 0)),
        ],
        out_specs=pl.BlockSpec((nb_sz, H, W, Cout), lambda nb: (nb, 0, 0, 0)),
        compiler_params=pltpu.CompilerParams(
            dimension_semantics=("parallel",),
            vmem_limit_bytes=_VMEM_LIMIT),
        cost_estimate=pl.CostEstimate(
            flops=N * Cin * HW + 2 * N * Cin * Cout,
            transcendentals=0,
            bytes_accessed=N * Cin * HW * itemsize
                           + N * Cout * HW * itemsize + Cin * Cout * 4),
    )(xt, w_folded, bias)

    return jnp.transpose(out_t, (0, 3, 1, 2))


# speedup vs baseline: 11.6950x; 1.0783x over previous
import functools

import jax
import jax.numpy as jnp
from jax.experimental import pallas as pl
from jax.experimental.pallas import tpu as pltpu

_BN_EPS = 1e-5
_VMEM_LIMIT = 48 * 1024 * 1024


def _fused_body(x_ref, w_ref, b_ref, o_ref):
    pooled = jnp.sum(x_ref[...], axis=(1, 2), dtype=jnp.float32)
    y = jax.lax.dot_general(pooled, w_ref[...],
                            (((1,), (1,)), ((), ())),
                            preferred_element_type=jnp.float32)
    act = jnp.maximum(y + b_ref[...], 0.0).astype(o_ref.dtype)
    o_ref[...] = jnp.broadcast_to(act[:, None, None, :], o_ref.shape)


def _largest_divisor(n, target):
    d = min(n, target)
    while n % d:
        d -= 1
    return d


def kernel(x, conv_w, gamma, beta, running_mean, running_var):
    N, Cin, H, W = x.shape
    Cout = conv_w.shape[0]
    HW = H * W

    scale = gamma.astype(jnp.float32) * jax.lax.rsqrt(
        running_var.astype(jnp.float32) + _BN_EPS)
    w2d = conv_w.reshape(Cout, Cin).astype(jnp.float32)
    w_folded = w2d * (scale[:, None] / HW)
    bias = (beta.astype(jnp.float32)
            - running_mean.astype(jnp.float32) * scale).reshape(1, Cout)

    xt = jnp.transpose(x, (0, 2, 3, 1))

    nb_sz = _largest_divisor(N, 1)
    n_nb = N // nb_sz
    itemsize = jnp.dtype(x.dtype).itemsize

    out_t = pl.pallas_call(
        _fused_body,
        out_shape=jax.ShapeDtypeStruct((N, H, W, Cout), x.dtype),
        grid=(n_nb,),
        in_specs=[
            pl.BlockSpec((nb_sz, H, W, Cin), lambda nb: (nb, 0, 0, 0)),
            pl.BlockSpec((Cout, Cin), lambda nb: (0, 0)),
            pl.BlockSpec((1, Cout), lambda nb: (0, 0)),
        ],
        out_specs=pl.BlockSpec((nb_sz, H, W, Cout), lambda nb: (nb, 0, 0, 0)),
        compiler_params=pltpu.CompilerParams(
            dimension_semantics=("parallel",),
            vmem_limit_bytes=_VMEM_LIMIT),
        cost_estimate=pl.CostEstimate(
            flops=N * Cin * HW + 2 * N * Cin * Cout,
            transcendentals=0,
            bytes_accessed=N * Cin * HW * itemsize
                           + N * Cout * HW * itemsize + Cin * Cout * 4),
    )(xt, w_folded, bias)

    return jnp.transpose(out_t, (0, 3, 1, 2))
